# baseline (speedup 1.0000x reference)
import functools
import math

import jax
import jax.numpy as jnp
from jax import lax
from jax.experimental import pallas as pl
from jax.experimental.pallas import tpu as pltpu
from jax.experimental.pallas import tpu_sc as plsc


_NC = 2
_NS = 16
_NW = _NC * _NS


def _sc_row_gather(table, idx):
    R, D = table.shape
    dt = table.dtype
    per_w = R // _NW
    chunk = 200
    nbuf = 4
    n_quads = per_w // (nbuf * chunk)
    n_tail = per_w // chunk - nbuf * n_quads

    mesh = plsc.VectorSubcoreMesh(core_axis_name="c", subcore_axis_name="s")

    @functools.partial(
        pl.kernel,
        out_type=jax.ShapeDtypeStruct((R, D), dt),
        mesh=mesh,
        scratch_types=[
            [pltpu.VMEM((chunk,), jnp.int32)] * nbuf,
            [pltpu.VMEM((chunk, D), dt)] * nbuf,
            [pltpu.SemaphoreType.DMA] * nbuf,
            [pltpu.SemaphoreType.DMA] * nbuf,
            pltpu.SemaphoreType.DMA,
        ],
    )
    def k(table_hbm, idx_hbm, out_hbm, idxs, rows, semi, semg, semo):
        wid = lax.axis_index("s") * _NC + lax.axis_index("c")
        wbase = wid * per_w

        def do_quad(bases, drain_prev):
            @pl.when(drain_prev)
            def _():
                for s in range(nbuf):
                    pltpu.make_async_copy(
                        rows[s], out_hbm.at[pl.ds(wbase, chunk)], semo).wait()
            cpi = [pltpu.make_async_copy(
                idx_hbm.at[pl.ds(bases[s], chunk)], idxs[s], semi[s])
                for s in range(nbuf)]
            for s in range(nbuf):
                cpi[s].start()
            cpg = [pltpu.make_async_copy(
                table_hbm.at[idxs[s]], rows[s], semg[s]) for s in range(nbuf)]
            for s in range(nbuf):
                cpi[s].wait()
                cpg[s].start()
            for s in range(nbuf):
                cpg[s].wait()
                pltpu.make_async_copy(
                    rows[s], out_hbm.at[pl.ds(bases[s], chunk)], semo).start()

        def body(q, carry):
            base0 = wbase + q * (nbuf * chunk)
            do_quad([base0 + s * chunk for s in range(nbuf)], q > 0)
            return carry

        lax.fori_loop(0, n_quads, body, 0)
        for s in range(nbuf):
            pltpu.make_async_copy(
                rows[s], out_hbm.at[pl.ds(wbase, chunk)], semo).wait()
        for t in range(n_tail):
            base = wbase + (nbuf * n_quads + t) * chunk
            pltpu.sync_copy(idx_hbm.at[pl.ds(base, chunk)], idxs[0])
            cp = pltpu.make_async_copy(table_hbm.at[idxs[0]], rows[0], semg[0])
            cp.start()
            cp.wait()
            pltpu.sync_copy(rows[0], out_hbm.at[pl.ds(base, chunk)])

    return k(table, idx)


def _sc_row_scatter(rows, idx):
    R, D = rows.shape
    dt = rows.dtype
    per_w = R // _NW
    chunk = 80
    nbuf = 4
    n_quads = per_w // (nbuf * chunk)
    n_tail = per_w // chunk - nbuf * n_quads

    mesh = plsc.VectorSubcoreMesh(core_axis_name="c", subcore_axis_name="s")

    @functools.partial(
        pl.kernel,
        out_type=jax.ShapeDtypeStruct((R, D), dt),
        mesh=mesh,
        scratch_types=[
            [pltpu.VMEM((chunk,), jnp.int32)] * nbuf,
            [pltpu.VMEM((chunk, D), dt)] * nbuf,
            [pltpu.SemaphoreType.DMA] * nbuf,
            pltpu.SemaphoreType.DMA,
        ],
    )
    def k(rows_hbm, idx_hbm, out_hbm, idxs, bufs, seml, sems):
        wid = lax.axis_index("s") * _NC + lax.axis_index("c")
        wbase = wid * per_w

        def do_quad(bases, drain_prev):
            @pl.when(drain_prev)
            def _():
                for s in range(nbuf):
                    pltpu.make_async_copy(bufs[s], out_hbm.at[idxs[s]],
                                          sems).wait()
            cps = []
            for s in range(nbuf):
                ci = pltpu.make_async_copy(
                    idx_hbm.at[pl.ds(bases[s], chunk)], idxs[s], seml[s])
                cr = pltpu.make_async_copy(
                    rows_hbm.at[pl.ds(bases[s], chunk)], bufs[s], seml[s])
                ci.start()
                cr.start()
                cps.append((ci, cr))
            for s in range(nbuf):
                cps[s][0].wait()
                cps[s][1].wait()
                pltpu.make_async_copy(bufs[s], out_hbm.at[idxs[s]],
                                      sems).start()

        def body(q, carry):
            base0 = wbase + q * (nbuf * chunk)
            do_quad([base0 + s * chunk for s in range(nbuf)], q > 0)
            return carry

        lax.fori_loop(0, n_quads, body, 0)
        for s in range(nbuf):
            pltpu.make_async_copy(bufs[s], out_hbm.at[idxs[s]], sems).wait()
        for t in range(n_tail):
            base = wbase + (nbuf * n_quads + t) * chunk
            pltpu.sync_copy(idx_hbm.at[pl.ds(base, chunk)], idxs[0])
            pltpu.sync_copy(rows_hbm.at[pl.ds(base, chunk)], bufs[0])
            cp = pltpu.make_async_copy(bufs[0], out_hbm.at[idxs[0]], sems)
            cp.start()
            cp.wait()

    return k(rows, idx)


_B = 200


def _pack_bf16(x):
    rows, hid = x.shape
    half = hid // 2
    b = x.astype(jnp.bfloat16)
    lof = b[:, :half].astype(jnp.float32)
    hif = b[:, half:].astype(jnp.float32)
    lo32 = lax.shift_right_logical(
        lax.bitcast_convert_type(lof, jnp.int32), 16)
    hi32 = lax.bitcast_convert_type(hif, jnp.int32) & jnp.int32(-65536)
    return hi32 | lo32


def _unpack_bf16(w):
    lo_f = lax.bitcast_convert_type(w << 16, jnp.float32)
    hi_f = lax.bitcast_convert_type(w & jnp.int32(-65536), jnp.float32)
    return jnp.concatenate([lo_f, hi_f], axis=-1)


def _mm(x, w):
    return jnp.dot(x.astype(jnp.bfloat16), w.astype(jnp.bfloat16),
                   preferred_element_type=jnp.float32)


def _head_group_matrix(hid, heads):
    dk = hid // heads
    r = lax.broadcasted_iota(jnp.int32, (hid, heads), 0)
    c = lax.broadcasted_iota(jnp.int32, (hid, heads), 1)
    return (r // dk == c).astype(jnp.float32)


def _rows_per_node(x, b, deg, hid):
    return jnp.broadcast_to(x[:, None, :], (b, deg, hid)).reshape(b * deg, hid)


def _attn_round_kernel(first_round, deg, heads, a_ref, d_ref, feat_ref,
                       wq, bq, wk, bk, wv, bv, wo, bo, wmp, bmp,
                       feat_out, x_out):
    b, hid = feat_ref.shape
    dk = hid // heads
    a = a_ref[...]
    if first_round:
        hcur = a
    else:
        hcur = jnp.maximum(a + d_ref[...], 0.0)
    feat = feat_ref[...]

    g = _head_group_matrix(hid, heads)
    q = (_mm(feat, wq[...]) + bq[...]) * (1.0 / math.sqrt(dk))
    kk = _mm(hcur, wk[...]) + bk[...]
    vv = _mm(hcur, wv[...]) + bv[...]

    qrep = _rows_per_node(q, b, deg, hid)
    s = _mm(kk * qrep, g)
    s3 = s.reshape(b, deg, heads)
    s3 = s3 - jnp.max(s3, axis=1, keepdims=True)
    e3 = jnp.exp(s3)
    p3 = e3 / jnp.sum(e3, axis=1, keepdims=True)
    p = p3.reshape(b * deg, heads)

    pv = _mm(p, g.T) * vv
    o = jnp.sum(pv.reshape(b, deg, hid), axis=1)
    feat_new = _mm(o, wo[...]) + bo[...] + feat

    pmp = _mm(feat_new, wmp[...]) + bmp[...]
    hw = _mm(hcur, wmp[...])
    feat_out[...] = feat_new
    x_out[...] = _rows_per_node(pmp, b, deg, hid) - hw


def _final_kernel(deg, a_ref, d_ref, feat_ref, xa_ref, wlast, blast, out_ref):
    b, hid = feat_ref.shape
    h2 = jnp.maximum(a_ref[...] + d_ref[...], 0.0)
    ms = jnp.sum(h2.reshape(b, deg, hid), axis=1)
    w0 = wlast[0:hid, :]
    w1 = wlast[hid:2 * hid, :]
    w2 = wlast[2 * hid:3 * hid, :]
    out_ref[...] = (
        _mm(ms, w0) + _mm(feat_ref[...], w1) + _mm(xa_ref[...], w2)
        + blast[...]
    )


def _full_spec(shape):
    n = len(shape)
    return pl.BlockSpec(shape, lambda i: (0,) * n)


def _attn_round(first_round, a_p, d_p, feat, weights, n, deg, hid, heads):
    wq, bq, wk, bk, wv, bv, wo, bo, wmp, bmp = weights
    b = _B
    grid = (n // b,)
    edge_spec = pl.BlockSpec((b * deg, hid), lambda i: (i, 0))
    node_spec = pl.BlockSpec((b, hid), lambda i: (i, 0))
    in_specs = [
        edge_spec, edge_spec, node_spec,
        _full_spec(wq.shape), _full_spec(bq.shape),
        _full_spec(wk.shape), _full_spec(bk.shape),
        _full_spec(wv.shape), _full_spec(bv.shape),
        _full_spec(wo.shape), _full_spec(bo.shape),
        _full_spec(wmp.shape), _full_spec(bmp.shape),
    ]
    out_shape = [
        jax.ShapeDtypeStruct((n, hid), jnp.float32),
        jax.ShapeDtypeStruct((n * deg, hid), jnp.float32),
    ]
    out_specs = [node_spec, edge_spec]
    return pl.pallas_call(
        functools.partial(_attn_round_kernel, first_round, deg, heads),
        grid=grid,
        in_specs=in_specs,
        out_specs=out_specs,
        out_shape=out_shape,
    )(a_p, d_p, feat, wq, bq, wk, bk, wv, bv, wo, bo, wmp, bmp)


def _final(a_p, d_p, feat, x_a, wlast, blast, n, deg, hid):
    b = _B
    grid = (n // b,)
    edge_spec = pl.BlockSpec((b * deg, hid), lambda i: (i, 0))
    node_spec = pl.BlockSpec((b, hid), lambda i: (i, 0))
    return pl.pallas_call(
        functools.partial(_final_kernel, deg),
        grid=grid,
        in_specs=[edge_spec, edge_spec, node_spec, node_spec,
                  _full_spec(wlast.shape), _full_spec(blast.shape)],
        out_specs=node_spec,
        out_shape=jax.ShapeDtypeStruct((n, hid), jnp.float32),
    )(a_p, d_p, feat, x_a, wlast, blast)


def kernel(x_a, edge_attr, Wq, bq, Wk, bk, Wv, bv, Wo, bo,
           Wmp0, bmp0, Wmp1, bmp1, Wlast, blast, edge_index, in_edges):
    n, hid = x_a.shape
    e = edge_attr.shape[0]
    deg = e // n
    heads = 4

    perm = in_edges.reshape(e).astype(jnp.int32)
    pxor = perm ^ 1

    bq2, bk2, bv2, bo2 = (x.reshape(1, hid) for x in (bq, bk, bv, bo))
    bmp0_2 = bmp0.reshape(1, hid)
    bmp1_2 = bmp1.reshape(1, hid)
    blast2 = blast.reshape(1, hid)

    a_p = _sc_row_gather(edge_attr, perm)

    zeros_d = a_p
    f1, x0 = _attn_round(True, a_p, zeros_d, x_a,
                         (Wq, bq2, Wk, bk2, Wv, bv2, Wo, bo2, Wmp0, bmp0_2),
                         n, deg, hid, heads)
    d0 = _sc_row_gather(_sc_row_scatter(x0, perm), pxor)
    f2, x1 = _attn_round(False, a_p, d0, f1,
                         (Wq, bq2, Wk, bk2, Wv, bv2, Wo, bo2, Wmp1, bmp1_2),
                         n, deg, hid, heads)
    d1 = _sc_row_gather(_sc_row_scatter(x1, perm), pxor)
    return _final(a_p, d1, f2, x_a, Wlast, blast2, n, deg, hid)


# speedup vs baseline: 1.0143x; 1.0143x over previous
import functools
import math

import jax
import jax.numpy as jnp
from jax import lax
from jax.experimental import pallas as pl
from jax.experimental.pallas import tpu as pltpu
from jax.experimental.pallas import tpu_sc as plsc


_NC = 2
_NS = 16
_NW = _NC * _NS


def _sc_row_gather(table, idx):
    R, D = table.shape
    dt = table.dtype
    per_w = R // _NW
    chunk = 200
    nbuf = 4
    n_quads = per_w // (nbuf * chunk)
    n_tail = per_w // chunk - nbuf * n_quads

    mesh = plsc.VectorSubcoreMesh(core_axis_name="c", subcore_axis_name="s")

    @functools.partial(
        pl.kernel,
        out_type=jax.ShapeDtypeStruct((R, D), dt),
        mesh=mesh,
        scratch_types=[
            [pltpu.VMEM((chunk,), jnp.int32)] * nbuf,
            [pltpu.VMEM((chunk, D), dt)] * nbuf,
            [pltpu.SemaphoreType.DMA] * nbuf,
            [pltpu.SemaphoreType.DMA] * nbuf,
            pltpu.SemaphoreType.DMA,
        ],
    )
    def k(table_hbm, idx_hbm, out_hbm, idxs, rows, semi, semg, semo):
        wid = lax.axis_index("s") * _NC + lax.axis_index("c")
        wbase = wid * per_w

        def do_quad(bases, drain_prev):
            @pl.when(drain_prev)
            def _():
                for s in range(nbuf):
                    pltpu.make_async_copy(
                        rows[s], out_hbm.at[pl.ds(wbase, chunk)], semo).wait()
            cpi = [pltpu.make_async_copy(
                idx_hbm.at[pl.ds(bases[s], chunk)], idxs[s], semi[s])
                for s in range(nbuf)]
            for s in range(nbuf):
                cpi[s].start()
            cpg = [pltpu.make_async_copy(
                table_hbm.at[idxs[s]], rows[s], semg[s]) for s in range(nbuf)]
            for s in range(nbuf):
                cpi[s].wait()
                cpg[s].start()
            for s in range(nbuf):
                cpg[s].wait()
                pltpu.make_async_copy(
                    rows[s], out_hbm.at[pl.ds(bases[s], chunk)], semo).start()

        def body(q, carry):
            base0 = wbase + q * (nbuf * chunk)
            do_quad([base0 + s * chunk for s in range(nbuf)], q > 0)
            return carry

        lax.fori_loop(0, n_quads, body, 0)
        for s in range(nbuf):
            pltpu.make_async_copy(
                rows[s], out_hbm.at[pl.ds(wbase, chunk)], semo).wait()
        for t in range(n_tail):
            base = wbase + (nbuf * n_quads + t) * chunk
            pltpu.sync_copy(idx_hbm.at[pl.ds(base, chunk)], idxs[0])
            cp = pltpu.make_async_copy(table_hbm.at[idxs[0]], rows[0], semg[0])
            cp.start()
            cp.wait()
            pltpu.sync_copy(rows[0], out_hbm.at[pl.ds(base, chunk)])

    return k(table, idx)


def _sc_row_scatter(rows, idx):
    R, D = rows.shape
    dt = rows.dtype
    per_w = R // _NW
    chunk = 80
    nbuf = 4
    n_quads = per_w // (nbuf * chunk)
    n_tail = per_w // chunk - nbuf * n_quads

    mesh = plsc.VectorSubcoreMesh(core_axis_name="c", subcore_axis_name="s")

    @functools.partial(
        pl.kernel,
        out_type=jax.ShapeDtypeStruct((R, D), dt),
        mesh=mesh,
        scratch_types=[
            [pltpu.VMEM((chunk,), jnp.int32)] * nbuf,
            [pltpu.VMEM((chunk, D), dt)] * nbuf,
            [pltpu.SemaphoreType.DMA] * nbuf,
            pltpu.SemaphoreType.DMA,
        ],
    )
    def k(rows_hbm, idx_hbm, out_hbm, idxs, bufs, seml, sems):
        wid = lax.axis_index("s") * _NC + lax.axis_index("c")
        wbase = wid * per_w

        def do_quad(bases, drain_prev):
            @pl.when(drain_prev)
            def _():
                for s in range(nbuf):
                    pltpu.make_async_copy(bufs[s], out_hbm.at[idxs[s]],
                                          sems).wait()
            cps = []
            for s in range(nbuf):
                ci = pltpu.make_async_copy(
                    idx_hbm.at[pl.ds(bases[s], chunk)], idxs[s], seml[s])
                cr = pltpu.make_async_copy(
                    rows_hbm.at[pl.ds(bases[s], chunk)], bufs[s], seml[s])
                ci.start()
                cr.start()
                cps.append((ci, cr))
            for s in range(nbuf):
                cps[s][0].wait()
                cps[s][1].wait()
                pltpu.make_async_copy(bufs[s], out_hbm.at[idxs[s]],
                                      sems).start()

        def body(q, carry):
            base0 = wbase + q * (nbuf * chunk)
            do_quad([base0 + s * chunk for s in range(nbuf)], q > 0)
            return carry

        lax.fori_loop(0, n_quads, body, 0)
        for s in range(nbuf):
            pltpu.make_async_copy(bufs[s], out_hbm.at[idxs[s]], sems).wait()
        for t in range(n_tail):
            base = wbase + (nbuf * n_quads + t) * chunk
            pltpu.sync_copy(idx_hbm.at[pl.ds(base, chunk)], idxs[0])
            pltpu.sync_copy(rows_hbm.at[pl.ds(base, chunk)], bufs[0])
            cp = pltpu.make_async_copy(bufs[0], out_hbm.at[idxs[0]], sems)
            cp.start()
            cp.wait()

    return k(rows, idx)


_B = 200


def _pack_bf16(x):
    rows, hid = x.shape
    half = hid // 2
    b = x.astype(jnp.bfloat16)
    lof = b[:, :half].astype(jnp.float32)
    hif = b[:, half:].astype(jnp.float32)
    lo32 = lax.shift_right_logical(
        lax.bitcast_convert_type(lof, jnp.int32), 16)
    hi32 = lax.bitcast_convert_type(hif, jnp.int32) & jnp.int32(-65536)
    return hi32 | lo32


def _unpack_bf16(w):
    lo_f = lax.bitcast_convert_type(w << 16, jnp.float32)
    hi_f = lax.bitcast_convert_type(w & jnp.int32(-65536), jnp.float32)
    return jnp.concatenate([lo_f, hi_f], axis=-1)


def _mm(x, w):
    return jnp.dot(x, w, preferred_element_type=jnp.float32)


def _head_group_matrix(hid, heads):
    dk = hid // heads
    r = lax.broadcasted_iota(jnp.int32, (hid, heads), 0)
    c = lax.broadcasted_iota(jnp.int32, (hid, heads), 1)
    return (r // dk == c).astype(jnp.float32)


def _rows_per_node(x, b, deg, hid):
    return jnp.broadcast_to(x[:, None, :], (b, deg, hid)).reshape(b * deg, hid)


def _attn_round_kernel(first_round, deg, heads, a_ref, d_ref, feat_ref,
                       wq, bq, wk, bk, wv, bv, wo, bo, wmp, bmp,
                       feat_out, x_out):
    b, hid = feat_ref.shape
    dk = hid // heads
    a = a_ref[...]
    if first_round:
        hcur = a
    else:
        hcur = jnp.maximum(a + d_ref[...], 0.0)
    feat = feat_ref[...]

    g = _head_group_matrix(hid, heads)
    q = (_mm(feat, wq[...]) + bq[...]) * (1.0 / math.sqrt(dk))
    kk = _mm(hcur, wk[...]) + bk[...]
    vv = _mm(hcur, wv[...]) + bv[...]

    qrep = _rows_per_node(q, b, deg, hid)
    s = _mm(kk * qrep, g)
    s3 = s.reshape(b, deg, heads)
    s3 = s3 - jnp.max(s3, axis=1, keepdims=True)
    e3 = jnp.exp(s3)
    p3 = e3 / jnp.sum(e3, axis=1, keepdims=True)
    p = p3.reshape(b * deg, heads)

    pv = _mm(p, g.T) * vv
    o = jnp.sum(pv.reshape(b, deg, hid), axis=1)
    feat_new = _mm(o, wo[...]) + bo[...] + feat

    pmp = _mm(feat_new, wmp[...]) + bmp[...]
    hw = _mm(hcur, wmp[...])
    feat_out[...] = feat_new
    x_out[...] = _rows_per_node(pmp, b, deg, hid) - hw


def _final_kernel(deg, a_ref, d_ref, feat_ref, xa_ref, wlast, blast, out_ref):
    b, hid = feat_ref.shape
    h2 = jnp.maximum(a_ref[...] + d_ref[...], 0.0)
    ms = jnp.sum(h2.reshape(b, deg, hid), axis=1)
    w0 = wlast[0:hid, :]
    w1 = wlast[hid:2 * hid, :]
    w2 = wlast[2 * hid:3 * hid, :]
    out_ref[...] = (
        _mm(ms, w0) + _mm(feat_ref[...], w1) + _mm(xa_ref[...], w2)
        + blast[...]
    )


def _full_spec(shape):
    n = len(shape)
    return pl.BlockSpec(shape, lambda i: (0,) * n)


def _attn_round(first_round, a_p, d_p, feat, weights, n, deg, hid, heads):
    wq, bq, wk, bk, wv, bv, wo, bo, wmp, bmp = weights
    b = _B
    grid = (n // b,)
    edge_spec = pl.BlockSpec((b * deg, hid), lambda i: (i, 0))
    node_spec = pl.BlockSpec((b, hid), lambda i: (i, 0))
    in_specs = [
        edge_spec, edge_spec, node_spec,
        _full_spec(wq.shape), _full_spec(bq.shape),
        _full_spec(wk.shape), _full_spec(bk.shape),
        _full_spec(wv.shape), _full_spec(bv.shape),
        _full_spec(wo.shape), _full_spec(bo.shape),
        _full_spec(wmp.shape), _full_spec(bmp.shape),
    ]
    out_shape = [
        jax.ShapeDtypeStruct((n, hid), jnp.float32),
        jax.ShapeDtypeStruct((n * deg, hid), jnp.float32),
    ]
    out_specs = [node_spec, edge_spec]
    return pl.pallas_call(
        functools.partial(_attn_round_kernel, first_round, deg, heads),
        grid=grid,
        in_specs=in_specs,
        out_specs=out_specs,
        out_shape=out_shape,
    )(a_p, d_p, feat, wq, bq, wk, bk, wv, bv, wo, bo, wmp, bmp)


def _final(a_p, d_p, feat, x_a, wlast, blast, n, deg, hid):
    b = _B
    grid = (n // b,)
    edge_spec = pl.BlockSpec((b * deg, hid), lambda i: (i, 0))
    node_spec = pl.BlockSpec((b, hid), lambda i: (i, 0))
    return pl.pallas_call(
        functools.partial(_final_kernel, deg),
        grid=grid,
        in_specs=[edge_spec, edge_spec, node_spec, node_spec,
                  _full_spec(wlast.shape), _full_spec(blast.shape)],
        out_specs=node_spec,
        out_shape=jax.ShapeDtypeStruct((n, hid), jnp.float32),
    )(a_p, d_p, feat, x_a, wlast, blast)


def kernel(x_a, edge_attr, Wq, bq, Wk, bk, Wv, bv, Wo, bo,
           Wmp0, bmp0, Wmp1, bmp1, Wlast, blast, edge_index, in_edges):
    n, hid = x_a.shape
    e = edge_attr.shape[0]
    deg = e // n
    heads = 4

    perm = in_edges.reshape(e).astype(jnp.int32)
    pxor = perm ^ 1

    bq2, bk2, bv2, bo2 = (x.reshape(1, hid) for x in (bq, bk, bv, bo))
    bmp0_2 = bmp0.reshape(1, hid)
    bmp1_2 = bmp1.reshape(1, hid)
    blast2 = blast.reshape(1, hid)

    a_p = _sc_row_gather(edge_attr, perm)

    zeros_d = a_p
    f1, x0 = _attn_round(True, a_p, zeros_d, x_a,
                         (Wq, bq2, Wk, bk2, Wv, bv2, Wo, bo2, Wmp0, bmp0_2),
                         n, deg, hid, heads)
    d0 = _sc_row_gather(_sc_row_scatter(x0, perm), pxor)
    f2, x1 = _attn_round(False, a_p, d0, f1,
                         (Wq, bq2, Wk, bk2, Wv, bv2, Wo, bo2, Wmp1, bmp1_2),
                         n, deg, hid, heads)
    d1 = _sc_row_gather(_sc_row_scatter(x1, perm), pxor)
    return _final(a_p, d1, f2, x_a, Wlast, blast2, n, deg, hid)


# speedup vs baseline: 1.0184x; 1.0040x over previous
import functools
import math

import jax
import jax.numpy as jnp
from jax import lax
from jax.experimental import pallas as pl
from jax.experimental.pallas import tpu as pltpu
from jax.experimental.pallas import tpu_sc as plsc


_NC = 2
_NS = 16
_NW = _NC * _NS


def _sc_row_gather(table, idx):
    R, D = table.shape
    dt = table.dtype
    per_w = R // _NW
    chunk = 80
    nbuf = 8
    n_quads = per_w // (nbuf * chunk)
    n_tail = per_w // chunk - nbuf * n_quads

    mesh = plsc.VectorSubcoreMesh(core_axis_name="c", subcore_axis_name="s")

    @functools.partial(
        pl.kernel,
        out_type=jax.ShapeDtypeStruct((R, D), dt),
        mesh=mesh,
        scratch_types=[
            [pltpu.VMEM((chunk,), jnp.int32)] * nbuf,
            [pltpu.VMEM((chunk, D), dt)] * nbuf,
            [pltpu.SemaphoreType.DMA] * nbuf,
            [pltpu.SemaphoreType.DMA] * nbuf,
            pltpu.SemaphoreType.DMA,
        ],
    )
    def k(table_hbm, idx_hbm, out_hbm, idxs, rows, semi, semg, semo):
        wid = lax.axis_index("s") * _NC + lax.axis_index("c")
        wbase = wid * per_w

        def do_quad(bases, drain_prev):
            @pl.when(drain_prev)
            def _():
                for s in range(nbuf):
                    pltpu.make_async_copy(
                        rows[s], out_hbm.at[pl.ds(wbase, chunk)], semo).wait()
            cpi = [pltpu.make_async_copy(
                idx_hbm.at[pl.ds(bases[s], chunk)], idxs[s], semi[s])
                for s in range(nbuf)]
            for s in range(nbuf):
                cpi[s].start()
            cpg = [pltpu.make_async_copy(
                table_hbm.at[idxs[s]], rows[s], semg[s]) for s in range(nbuf)]
            for s in range(nbuf):
                cpi[s].wait()
                cpg[s].start()
            for s in range(nbuf):
                cpg[s].wait()
                pltpu.make_async_copy(
                    rows[s], out_hbm.at[pl.ds(bases[s], chunk)], semo).start()

        def body(q, carry):
            base0 = wbase + q * (nbuf * chunk)
            do_quad([base0 + s * chunk for s in range(nbuf)], q > 0)
            return carry

        lax.fori_loop(0, n_quads, body, 0)
        for s in range(nbuf):
            pltpu.make_async_copy(
                rows[s], out_hbm.at[pl.ds(wbase, chunk)], semo).wait()
        for t in range(n_tail):
            base = wbase + (nbuf * n_quads + t) * chunk
            pltpu.sync_copy(idx_hbm.at[pl.ds(base, chunk)], idxs[0])
            cp = pltpu.make_async_copy(table_hbm.at[idxs[0]], rows[0], semg[0])
            cp.start()
            cp.wait()
            pltpu.sync_copy(rows[0], out_hbm.at[pl.ds(base, chunk)])

    return k(table, idx)


def _sc_row_scatter(rows, idx):
    R, D = rows.shape
    dt = rows.dtype
    per_w = R // _NW
    chunk = 80
    nbuf = 8
    n_quads = per_w // (nbuf * chunk)
    n_tail = per_w // chunk - nbuf * n_quads

    mesh = plsc.VectorSubcoreMesh(core_axis_name="c", subcore_axis_name="s")

    @functools.partial(
        pl.kernel,
        out_type=jax.ShapeDtypeStruct((R, D), dt),
        mesh=mesh,
        scratch_types=[
            [pltpu.VMEM((chunk,), jnp.int32)] * nbuf,
            [pltpu.VMEM((chunk, D), dt)] * nbuf,
            [pltpu.SemaphoreType.DMA] * nbuf,
            pltpu.SemaphoreType.DMA,
        ],
    )
    def k(rows_hbm, idx_hbm, out_hbm, idxs, bufs, seml, sems):
        wid = lax.axis_index("s") * _NC + lax.axis_index("c")
        wbase = wid * per_w

        def do_quad(bases, drain_prev):
            @pl.when(drain_prev)
            def _():
                for s in range(nbuf):
                    pltpu.make_async_copy(bufs[s], out_hbm.at[idxs[s]],
                                          sems).wait()
            cps = []
            for s in range(nbuf):
                ci = pltpu.make_async_copy(
                    idx_hbm.at[pl.ds(bases[s], chunk)], idxs[s], seml[s])
                cr = pltpu.make_async_copy(
                    rows_hbm.at[pl.ds(bases[s], chunk)], bufs[s], seml[s])
                ci.start()
                cr.start()
                cps.append((ci, cr))
            for s in range(nbuf):
                cps[s][0].wait()
                cps[s][1].wait()
                pltpu.make_async_copy(bufs[s], out_hbm.at[idxs[s]],
                                      sems).start()

        def body(q, carry):
            base0 = wbase + q * (nbuf * chunk)
            do_quad([base0 + s * chunk for s in range(nbuf)], q > 0)
            return carry

        lax.fori_loop(0, n_quads, body, 0)
        for s in range(nbuf):
            pltpu.make_async_copy(bufs[s], out_hbm.at[idxs[s]], sems).wait()
        for t in range(n_tail):
            base = wbase + (nbuf * n_quads + t) * chunk
            pltpu.sync_copy(idx_hbm.at[pl.ds(base, chunk)], idxs[0])
            pltpu.sync_copy(rows_hbm.at[pl.ds(base, chunk)], bufs[0])
            cp = pltpu.make_async_copy(bufs[0], out_hbm.at[idxs[0]], sems)
            cp.start()
            cp.wait()

    return k(rows, idx)


_B = 200


def _pack_bf16(x):
    rows, hid = x.shape
    half = hid // 2
    b = x.astype(jnp.bfloat16)
    lof = b[:, :half].astype(jnp.float32)
    hif = b[:, half:].astype(jnp.float32)
    lo32 = lax.shift_right_logical(
        lax.bitcast_convert_type(lof, jnp.int32), 16)
    hi32 = lax.bitcast_convert_type(hif, jnp.int32) & jnp.int32(-65536)
    return hi32 | lo32


def _unpack_bf16(w):
    lo_f = lax.bitcast_convert_type(w << 16, jnp.float32)
    hi_f = lax.bitcast_convert_type(w & jnp.int32(-65536), jnp.float32)
    return jnp.concatenate([lo_f, hi_f], axis=-1)


def _mm(x, w):
    return jnp.dot(x, w, preferred_element_type=jnp.float32)


def _head_group_matrix(hid, heads):
    dk = hid // heads
    r = lax.broadcasted_iota(jnp.int32, (hid, heads), 0)
    c = lax.broadcasted_iota(jnp.int32, (hid, heads), 1)
    return (r // dk == c).astype(jnp.float32)


def _rows_per_node(x, b, deg, hid):
    return jnp.broadcast_to(x[:, None, :], (b, deg, hid)).reshape(b * deg, hid)


def _attn_round_kernel(first_round, deg, heads, a_ref, d_ref, feat_ref,
                       wq, bq, wk, bk, wv, bv, wo, bo, wmp, bmp,
                       feat_out, x_out):
    b, hid = feat_ref.shape
    dk = hid // heads
    a = a_ref[...]
    if first_round:
        hcur = a
    else:
        hcur = jnp.maximum(a + d_ref[...], 0.0)
    feat = feat_ref[...]

    g = _head_group_matrix(hid, heads)
    q = (_mm(feat, wq[...]) + bq[...]) * (1.0 / math.sqrt(dk))
    kk = _mm(hcur, wk[...]) + bk[...]
    vv = _mm(hcur, wv[...]) + bv[...]

    qrep = _rows_per_node(q, b, deg, hid)
    s = _mm(kk * qrep, g)
    s3 = s.reshape(b, deg, heads)
    s3 = s3 - jnp.max(s3, axis=1, keepdims=True)
    e3 = jnp.exp(s3)
    p3 = e3 / jnp.sum(e3, axis=1, keepdims=True)
    p = p3.reshape(b * deg, heads)

    pv = _mm(p, g.T) * vv
    o = jnp.sum(pv.reshape(b, deg, hid), axis=1)
    feat_new = _mm(o, wo[...]) + bo[...] + feat

    pmp = _mm(feat_new, wmp[...]) + bmp[...]
    hw = _mm(hcur, wmp[...])
    feat_out[...] = feat_new
    x_out[...] = _rows_per_node(pmp, b, deg, hid) - hw


def _final_kernel(deg, a_ref, d_ref, feat_ref, xa_ref, wlast, blast, out_ref):
    b, hid = feat_ref.shape
    h2 = jnp.maximum(a_ref[...] + d_ref[...], 0.0)
    ms = jnp.sum(h2.reshape(b, deg, hid), axis=1)
    w0 = wlast[0:hid, :]
    w1 = wlast[hid:2 * hid, :]
    w2 = wlast[2 * hid:3 * hid, :]
    out_ref[...] = (
        _mm(ms, w0) + _mm(feat_ref[...], w1) + _mm(xa_ref[...], w2)
        + blast[...]
    )


def _full_spec(shape):
    n = len(shape)
    return pl.BlockSpec(shape, lambda i: (0,) * n)


def _attn_round(first_round, a_p, d_p, feat, weights, n, deg, hid, heads):
    wq, bq, wk, bk, wv, bv, wo, bo, wmp, bmp = weights
    b = _B
    grid = (n // b,)
    edge_spec = pl.BlockSpec((b * deg, hid), lambda i: (i, 0))
    node_spec = pl.BlockSpec((b, hid), lambda i: (i, 0))
    in_specs = [
        edge_spec, edge_spec, node_spec,
        _full_spec(wq.shape), _full_spec(bq.shape),
        _full_spec(wk.shape), _full_spec(bk.shape),
        _full_spec(wv.shape), _full_spec(bv.shape),
        _full_spec(wo.shape), _full_spec(bo.shape),
        _full_spec(wmp.shape), _full_spec(bmp.shape),
    ]
    out_shape = [
        jax.ShapeDtypeStruct((n, hid), jnp.float32),
        jax.ShapeDtypeStruct((n * deg, hid), jnp.float32),
    ]
    out_specs = [node_spec, edge_spec]
    return pl.pallas_call(
        functools.partial(_attn_round_kernel, first_round, deg, heads),
        grid=grid,
        in_specs=in_specs,
        out_specs=out_specs,
        out_shape=out_shape,
    )(a_p, d_p, feat, wq, bq, wk, bk, wv, bv, wo, bo, wmp, bmp)


def _final(a_p, d_p, feat, x_a, wlast, blast, n, deg, hid):
    b = _B
    grid = (n // b,)
    edge_spec = pl.BlockSpec((b * deg, hid), lambda i: (i, 0))
    node_spec = pl.BlockSpec((b, hid), lambda i: (i, 0))
    return pl.pallas_call(
        functools.partial(_final_kernel, deg),
        grid=grid,
        in_specs=[edge_spec, edge_spec, node_spec, node_spec,
                  _full_spec(wlast.shape), _full_spec(blast.shape)],
        out_specs=node_spec,
        out_shape=jax.ShapeDtypeStruct((n, hid), jnp.float32),
    )(a_p, d_p, feat, x_a, wlast, blast)


def kernel(x_a, edge_attr, Wq, bq, Wk, bk, Wv, bv, Wo, bo,
           Wmp0, bmp0, Wmp1, bmp1, Wlast, blast, edge_index, in_edges):
    n, hid = x_a.shape
    e = edge_attr.shape[0]
    deg = e // n
    heads = 4

    perm = in_edges.reshape(e).astype(jnp.int32)
    pxor = perm ^ 1

    bq2, bk2, bv2, bo2 = (x.reshape(1, hid) for x in (bq, bk, bv, bo))
    bmp0_2 = bmp0.reshape(1, hid)
    bmp1_2 = bmp1.reshape(1, hid)
    blast2 = blast.reshape(1, hid)

    a_p = _sc_row_gather(edge_attr, perm)

    zeros_d = a_p
    f1, x0 = _attn_round(True, a_p, zeros_d, x_a,
                         (Wq, bq2, Wk, bk2, Wv, bv2, Wo, bo2, Wmp0, bmp0_2),
                         n, deg, hid, heads)
    d0 = _sc_row_gather(_sc_row_scatter(x0, perm), pxor)
    f2, x1 = _attn_round(False, a_p, d0, f1,
                         (Wq, bq2, Wk, bk2, Wv, bv2, Wo, bo2, Wmp1, bmp1_2),
                         n, deg, hid, heads)
    d1 = _sc_row_gather(_sc_row_scatter(x1, perm), pxor)
    return _final(a_p, d1, f2, x_a, Wlast, blast2, n, deg, hid)


# speedup vs baseline: 1.0190x; 1.0006x over previous
import functools
import math

import jax
import jax.numpy as jnp
from jax import lax
from jax.experimental import pallas as pl
from jax.experimental.pallas import tpu as pltpu
from jax.experimental.pallas import tpu_sc as plsc


_NC = 2
_NS = 16
_NW = _NC * _NS


def _sc_row_gather(table, idx):
    R, D = table.shape
    dt = table.dtype
    per_w = R // _NW
    chunk = 80
    nbuf = 8
    n_quads = per_w // (nbuf * chunk)
    n_tail = per_w // chunk - nbuf * n_quads

    mesh = plsc.VectorSubcoreMesh(core_axis_name="c", subcore_axis_name="s")

    @functools.partial(
        pl.kernel,
        out_type=jax.ShapeDtypeStruct((R, D), dt),
        mesh=mesh,
        scratch_types=[
            [pltpu.VMEM((chunk,), jnp.int32)] * nbuf,
            [pltpu.VMEM((chunk, D), dt)] * nbuf,
            [pltpu.SemaphoreType.DMA] * nbuf,
            [pltpu.SemaphoreType.DMA] * nbuf,
            pltpu.SemaphoreType.DMA,
        ],
    )
    def k(table_hbm, idx_hbm, out_hbm, idxs, rows, semi, semg, semo):
        wid = lax.axis_index("s") * _NC + lax.axis_index("c")
        wbase = wid * per_w

        def do_quad(bases, drain_prev):
            @pl.when(drain_prev)
            def _():
                for s in range(nbuf):
                    pltpu.make_async_copy(
                        rows[s], out_hbm.at[pl.ds(wbase, chunk)], semo).wait()
            cpi = [pltpu.make_async_copy(
                idx_hbm.at[pl.ds(bases[s], chunk)], idxs[s], semi[s])
                for s in range(nbuf)]
            for s in range(nbuf):
                cpi[s].start()
            cpg = [pltpu.make_async_copy(
                table_hbm.at[idxs[s]], rows[s], semg[s]) for s in range(nbuf)]
            for s in range(nbuf):
                cpi[s].wait()
                cpg[s].start()
            for s in range(nbuf):
                cpg[s].wait()
                pltpu.make_async_copy(
                    rows[s], out_hbm.at[pl.ds(bases[s], chunk)], semo).start()

        def body(q, carry):
            base0 = wbase + q * (nbuf * chunk)
            do_quad([base0 + s * chunk for s in range(nbuf)], q > 0)
            return carry

        lax.fori_loop(0, n_quads, body, 0)
        for s in range(nbuf):
            pltpu.make_async_copy(
                rows[s], out_hbm.at[pl.ds(wbase, chunk)], semo).wait()
        for t in range(n_tail):
            base = wbase + (nbuf * n_quads + t) * chunk
            pltpu.sync_copy(idx_hbm.at[pl.ds(base, chunk)], idxs[0])
            cp = pltpu.make_async_copy(table_hbm.at[idxs[0]], rows[0], semg[0])
            cp.start()
            cp.wait()
            pltpu.sync_copy(rows[0], out_hbm.at[pl.ds(base, chunk)])

    return k(table, idx)


def _sc_row_scatter(rows, idx):
    R, D = rows.shape
    dt = rows.dtype
    per_w = R // _NW
    chunk = 80
    nbuf = 8
    n_quads = per_w // (nbuf * chunk)
    n_tail = per_w // chunk - nbuf * n_quads

    mesh = plsc.VectorSubcoreMesh(core_axis_name="c", subcore_axis_name="s")

    @functools.partial(
        pl.kernel,
        out_type=jax.ShapeDtypeStruct((R, D), dt),
        mesh=mesh,
        scratch_types=[
            [pltpu.VMEM((chunk,), jnp.int32)] * nbuf,
            [pltpu.VMEM((chunk, D), dt)] * nbuf,
            [pltpu.SemaphoreType.DMA] * nbuf,
            pltpu.SemaphoreType.DMA,
        ],
    )
    def k(rows_hbm, idx_hbm, out_hbm, idxs, bufs, seml, sems):
        wid = lax.axis_index("s") * _NC + lax.axis_index("c")
        wbase = wid * per_w

        def do_quad(bases, drain_prev):
            @pl.when(drain_prev)
            def _():
                for s in range(nbuf):
                    pltpu.make_async_copy(bufs[s], out_hbm.at[idxs[s]],
                                          sems).wait()
            cps = []
            for s in range(nbuf):
                ci = pltpu.make_async_copy(
                    idx_hbm.at[pl.ds(bases[s], chunk)], idxs[s], seml[s])
                cr = pltpu.make_async_copy(
                    rows_hbm.at[pl.ds(bases[s], chunk)], bufs[s], seml[s])
                ci.start()
                cr.start()
                cps.append((ci, cr))
            for s in range(nbuf):
                cps[s][0].wait()
                cps[s][1].wait()
                pltpu.make_async_copy(bufs[s], out_hbm.at[idxs[s]],
                                      sems).start()

        def body(q, carry):
            base0 = wbase + q * (nbuf * chunk)
            do_quad([base0 + s * chunk for s in range(nbuf)], q > 0)
            return carry

        lax.fori_loop(0, n_quads, body, 0)
        for s in range(nbuf):
            pltpu.make_async_copy(bufs[s], out_hbm.at[idxs[s]], sems).wait()
        for t in range(n_tail):
            base = wbase + (nbuf * n_quads + t) * chunk
            pltpu.sync_copy(idx_hbm.at[pl.ds(base, chunk)], idxs[0])
            pltpu.sync_copy(rows_hbm.at[pl.ds(base, chunk)], bufs[0])
            cp = pltpu.make_async_copy(bufs[0], out_hbm.at[idxs[0]], sems)
            cp.start()
            cp.wait()

    return k(rows, idx)


_B = 200


def _mm(x, w):
    return jnp.dot(x, w, preferred_element_type=jnp.float32)


def _head_group_matrix(hid, heads):
    dk = hid // heads
    r = lax.broadcasted_iota(jnp.int32, (hid, heads), 0)
    c = lax.broadcasted_iota(jnp.int32, (hid, heads), 1)
    return (r // dk == c).astype(jnp.float32)


def _rows_per_node(x, b, deg, hid):
    return jnp.broadcast_to(x[:, None, :], (b, deg, hid)).reshape(b * deg, hid)


def _attn_round_kernel(first_round, deg, heads, a_ref, d_ref, feat_ref,
                       wq, bq, wk, bk, wv, bv, wo, bo, wmp, bmp,
                       feat_out, x_out):
    b, hid = feat_ref.shape
    dk = hid // heads
    a = a_ref[...]
    if first_round:
        hcur = a
    else:
        hcur = jnp.maximum(a + d_ref[...], 0.0)
    feat = feat_ref[...]

    g = _head_group_matrix(hid, heads)
    q = (_mm(feat, wq[...]) + bq[...]) * (1.0 / math.sqrt(dk))
    kk = _mm(hcur, wk[...]) + bk[...]
    vv = _mm(hcur, wv[...]) + bv[...]

    qrep = _rows_per_node(q, b, deg, hid)
    s = _mm(kk * qrep, g)
    s3 = s.reshape(b, deg, heads)
    s3 = s3 - jnp.max(s3, axis=1, keepdims=True)
    e3 = jnp.exp(s3)
    p3 = e3 / jnp.sum(e3, axis=1, keepdims=True)
    p = p3.reshape(b * deg, heads)

    pv = _mm(p, g.T) * vv
    o = jnp.sum(pv.reshape(b, deg, hid), axis=1)
    feat_new = _mm(o, wo[...]) + bo[...] + feat

    pmp = _mm(feat_new, wmp[...]) + bmp[...]
    hw = _mm(hcur, wmp[...])
    feat_out[...] = feat_new
    x_out[...] = _rows_per_node(pmp, b, deg, hid) - hw


def _final_kernel(deg, a_ref, d_ref, feat_ref, xa_ref, wlast, blast, out_ref):
    b, hid = feat_ref.shape
    h2 = jnp.maximum(a_ref[...] + d_ref[...], 0.0)
    ms = jnp.sum(h2.reshape(b, deg, hid), axis=1)
    w0 = wlast[0:hid, :]
    w1 = wlast[hid:2 * hid, :]
    w2 = wlast[2 * hid:3 * hid, :]
    out_ref[...] = (
        _mm(ms, w0) + _mm(feat_ref[...], w1) + _mm(xa_ref[...], w2)
        + blast[...]
    )


def _full_spec(shape):
    n = len(shape)
    return pl.BlockSpec(shape, lambda i: (0,) * n)


def _attn_round(first_round, a_p, d_p, feat, weights, n, deg, hid, heads):
    wq, bq, wk, bk, wv, bv, wo, bo, wmp, bmp = weights
    b = _B
    grid = (n // b,)
    edge_spec = pl.BlockSpec((b * deg, hid), lambda i: (i, 0))
    node_spec = pl.BlockSpec((b, hid), lambda i: (i, 0))
    in_specs = [
        edge_spec, edge_spec, node_spec,
        _full_spec(wq.shape), _full_spec(bq.shape),
        _full_spec(wk.shape), _full_spec(bk.shape),
        _full_spec(wv.shape), _full_spec(bv.shape),
        _full_spec(wo.shape), _full_spec(bo.shape),
        _full_spec(wmp.shape), _full_spec(bmp.shape),
    ]
    out_shape = [
        jax.ShapeDtypeStruct((n, hid), jnp.float32),
        jax.ShapeDtypeStruct((n * deg, hid), jnp.float32),
    ]
    out_specs = [node_spec, edge_spec]
    return pl.pallas_call(
        functools.partial(_attn_round_kernel, first_round, deg, heads),
        grid=grid,
        in_specs=in_specs,
        out_specs=out_specs,
        out_shape=out_shape,
    )(a_p, d_p, feat, wq, bq, wk, bk, wv, bv, wo, bo, wmp, bmp)


def _final(a_p, d_p, feat, x_a, wlast, blast, n, deg, hid):
    b = _B
    grid = (n // b,)
    edge_spec = pl.BlockSpec((b * deg, hid), lambda i: (i, 0))
    node_spec = pl.BlockSpec((b, hid), lambda i: (i, 0))
    return pl.pallas_call(
        functools.partial(_final_kernel, deg),
        grid=grid,
        in_specs=[edge_spec, edge_spec, node_spec, node_spec,
                  _full_spec(wlast.shape), _full_spec(blast.shape)],
        out_specs=node_spec,
        out_shape=jax.ShapeDtypeStruct((n, hid), jnp.float32),
    )(a_p, d_p, feat, x_a, wlast, blast)


def kernel(x_a, edge_attr, Wq, bq, Wk, bk, Wv, bv, Wo, bo,
           Wmp0, bmp0, Wmp1, bmp1, Wlast, blast, edge_index, in_edges):
    n, hid = x_a.shape
    e = edge_attr.shape[0]
    deg = e // n
    heads = 4

    perm = in_edges.reshape(e).astype(jnp.int32)
    pxor = perm ^ 1

    bq2, bk2, bv2, bo2 = (x.reshape(1, hid) for x in (bq, bk, bv, bo))
    bmp0_2 = bmp0.reshape(1, hid)
    bmp1_2 = bmp1.reshape(1, hid)
    blast2 = blast.reshape(1, hid)

    a_p = _sc_row_gather(edge_attr, perm)

    zeros_d = a_p
    f1, x0 = _attn_round(True, a_p, zeros_d, x_a,
                         (Wq, bq2, Wk, bk2, Wv, bv2, Wo, bo2, Wmp0, bmp0_2),
                         n, deg, hid, heads)
    d0 = _sc_row_gather(_sc_row_scatter(x0, perm), pxor)
    f2, x1 = _attn_round(False, a_p, d0, f1,
                         (Wq, bq2, Wk, bk2, Wv, bv2, Wo, bo2, Wmp1, bmp1_2),
                         n, deg, hid, heads)
    d1 = _sc_row_gather(_sc_row_scatter(x1, perm), pxor)
    return _final(a_p, d1, f2, x_a, Wlast, blast2, n, deg, hid)
